# baseline (speedup 1.0000x reference)
import jax
import jax.numpy as jnp
import numpy as np
from jax.experimental import pallas as pl

_B, _N, _INP, _HIDE, _P = 4, 4096, 768, 64, 2
_EPS = 1e-6
_TK = 2048
_T = _B * _N
_NT = _T // _TK
_TPB = _N // _TK

_consts_cache = []


def _build_consts():
    key = jax.random.key(42)
    noise = jax.random.normal(
        jax.random.fold_in(key, 0), (_P, _B, _N, _HIDE), jnp.float32)
    g = jax.random.gumbel(
        jax.random.fold_in(key, 1), (_P, _B * _N * _HIDE, _P), jnp.float32)
    d = (g[..., 0] - g[..., 1]).reshape(_P, _T, _HIDE)
    eps = jax.random.normal(
        jax.random.fold_in(key, 99), (_B, _N, _HIDE),
        jnp.float32).reshape(_T, _HIDE)
    npack = jnp.concatenate(
        [noise[0].reshape(_T, _HIDE), noise[1].reshape(_T, _HIDE)], axis=1)
    dpack = jnp.concatenate([d[0], d[1]], axis=1)
    sh = jnp.asarray(np.block(
        [[np.ones((_HIDE, _HIDE), np.float32),
          np.zeros((_HIDE, _HIDE), np.float32)],
         [np.zeros((_HIDE, _HIDE), np.float32),
          np.ones((_HIDE, _HIDE), np.float32)]]))
    eh = jnp.asarray(np.concatenate(
        [np.zeros((1, _HIDE), np.float32),
         np.full((1, _HIDE), _EPS, np.float32)], axis=1))
    return npack, dpack, eps, sh, eh


def _consts():
    if _consts_cache:
        return _consts_cache[0]
    try:
        with jax.ensure_compile_time_eval():
            vals = tuple(map(jnp.asarray, _build_consts()))
        _consts_cache.append(vals)
        return vals
    except Exception:
        return _build_consts()


def _dot(a, b):
    return jax.lax.dot_general(a, b, (((1,), (0,)), ((), ())),
                               preferred_element_type=jnp.float32)


def _pass1_body(x1r, x2r, npr, dpr,
                war, bar, wbr, bbr,
                wru1r, bru1r, wc1r, bc1r,
                wru2r, bru2r, wc2r, bc2r,
                wp1r, bp1r, wmv1r, bmv1r,
                wp2r, bp2r, wmv2r, bmv2r,
                wfvr, bfvr, shr, ehr,
                fmfv_o, var_o, ps_o):
    h = _HIDE

    def lo(z):
        return z[:, :h]

    def hi(z):
        return z[:, h:]

    def rot(z):
        return jnp.concatenate([z[:, h:], z[:, :h]], axis=1)

    def dlo(z):
        return jnp.concatenate([z[:, :h], z[:, :h]], axis=1)

    def dhi(z):
        return jnp.concatenate([z[:, h:], z[:, h:]], axis=1)

    ab1 = _dot(x1r[...], war[...]) + bar[...]
    ab2 = _dot(x2r[...], wbr[...]) + bbr[...]
    a1, b2 = lo(ab1), hi(ab1)
    b1, a2 = lo(ab2), hi(ab2)

    def gate(a, b, wru, bru, wc, bc):
        comb = jnp.concatenate([a, b], axis=1)
        ru = jax.nn.sigmoid(_dot(comb, wru[...]) + bru[...])
        r, u = lo(ru), hi(ru)
        cand = jnp.tanh(_dot(jnp.concatenate([r * a, b], axis=1), wc[...])
                        + bc[...])
        return u * cand + (1.0 - u) * a

    feat1 = gate(a1, b1, wru1r, bru1r, wc1r, bc1r)
    feat2 = gate(a2, b2, wru2r, bru2r, wc2r, bc2r)

    h1 = jnp.maximum(_dot(feat1, wp1r[...]) + bp1r[...], 0.0)
    mv1 = _dot(h1, wmv1r[...]) + bmv1r[...]
    h2 = jnp.maximum(_dot(feat2, wp2r[...]) + bp2r[...], 0.0)
    mv2 = _dot(h2, wmv2r[...]) + bmv2r[...]

    eh = ehr[...]
    rec1 = 1.0 / jnp.maximum(mv1, _EPS)
    rec2 = 1.0 / jnp.maximum(mv2, _EPS)
    wf = 1.0 / (rec1 + rec2)
    meanvar = _dot(wf, wfvr[...]) + bfvr[...]

    mcvc = jnp.maximum((mv1 + mv2) * 0.5, _EPS)
    ss = jnp.sqrt(mcvc + eh)
    std2 = dhi(jnp.maximum(ss, _EPS))
    parts = dlo(mcvc) + std2 * npr[...]
    me2 = dlo(jnp.maximum(mv2, _EPS))
    rve2 = dhi(rec2)
    d = parts - me2
    dq = (d * d) * rve2
    q2 = _dot(dq, shr[...])
    wu2 = jnp.maximum(jnp.exp(-0.5 * q2), _EPS)
    s2 = (wu2 + rot(wu2)) * 0.5
    w2 = jnp.maximum(wu2 * 0.5 / s2, _EPS)
    lw = jnp.log(w2)
    t2 = dlo(rot(lw) - lw)
    idx = t2 > dpr[...]
    pn = jnp.where(idx, dhi(parts), dlo(parts))
    sw2 = w2 + rot(w2)
    wpn = w2 * pn
    fm2 = (wpn + rot(wpn)) / sw2
    df = pn - fm2
    fvt = w2 * (df * df)
    fv2 = (fvt + rot(fvt)) / sw2
    thr2 = meanvar * rot(jnp.sqrt(wf))
    cond = jnp.abs(fm2 - dlo(wf)) > dlo(thr2)
    sub = wf + eh
    fmfv = jnp.where(cond, sub,
                     jnp.concatenate([lo(fm2), hi(fv2)], axis=1))
    fvl = jnp.log(fmfv + eh)

    fmfv_o[...] = jnp.concatenate([lo(fmfv), hi(fvl)], axis=1)
    var_o[...] = hi(meanvar)
    pspack = jnp.concatenate([lo(rot(fvl)), hi(meanvar)], axis=1)
    ps_o[...] = jnp.sum(pspack, axis=0, keepdims=True).reshape(1, 1, 2 * h)


def _pass2_body(fmfvr, varr, epsr, psr, qwr, qbr, wpbr, bpbr, out_o):
    h = _HIDE
    b = pl.program_id(0) // _TPB
    ps = psr[...].reshape(_NT, 2 * h)
    rows = jax.lax.broadcasted_iota(jnp.int32, (_NT, 1), 0)
    mask = (rows // _TPB) == b
    mean_row = jnp.sum(jnp.where(mask, ps, 0.0), axis=0, keepdims=True) / _N
    qs = _dot(mean_row, qwr[...]) + qbr[...]
    q0, q1 = qs[0, 0], qs[0, 1]
    mx = jnp.maximum(q0, q1)
    e0, e1 = jnp.exp(q0 - mx), jnp.exp(q1 - mx)
    w0 = e0 / (e0 + e1)
    w1 = e1 / (e0 + e1)
    fmfv = fmfvr[...]
    fvc = w0 * fmfv[:, h:] + w1 * varr[...]
    fused = epsr[...] * jnp.exp(0.5 * fvc) + fmfv[:, :h]
    out_o[...] = _dot(fused, wpbr[...]) + bpbr[...]


def _tok_spec(width):
    return pl.BlockSpec((_TK, width), lambda i: (i, 0))


def _rep_spec(shape):
    nd = len(shape)
    return pl.BlockSpec(shape, lambda i, _n=nd: (0,) * _n)


def _run(x1, x2, params, interpret=False):
    p = params
    npack, dpack, eps, sh, eh = _consts()
    cat = jnp.concatenate
    h = _HIDE
    z64 = jnp.zeros((h, h), jnp.float32)
    wa = cat([p["g1_p1_w"], p["g2_p2_w"]], 1)
    ba = cat([p["g1_p1_b"], p["g2_p2_b"]])[None]
    wb = cat([p["g1_p2_w"], p["g2_p1_w"]], 1)
    bb = cat([p["g1_p2_b"], p["g2_p1_b"]])[None]
    wru1 = cat([p["g1_r_w"], p["g1_u_w"]], 1)
    bru1 = cat([p["g1_r_b"], p["g1_u_b"]])[None]
    wru2 = cat([p["g2_r_w"], p["g2_u_w"]], 1)
    bru2 = cat([p["g2_r_b"], p["g2_u_b"]])[None]
    wmv1 = cat([p["fcmean1_w"], p["fcvar1_w"]], 1)
    bmv1 = cat([p["fcmean1_b"], p["fcvar1_b"]])[None]
    wmv2 = cat([p["fcmean2_w"], p["fcvar2_w"]], 1)
    bmv2 = cat([p["fcmean2_b"], p["fcvar2_b"]])[None]
    wfv = cat([cat([p["fuse_mean_w"], z64], 1),
               cat([z64, p["fuse_var_w"]], 1)], 0)
    bfv = cat([p["fuse_mean_b"], p["fuse_var_b"]])[None]
    qw = jnp.pad(p["qe_w"], ((0, 0), (0, 6)))
    qb = jnp.pad(p["qe_b"], (0, 6))[None]

    f32 = jnp.float32
    fmfv, var, ps = pl.pallas_call(
        _pass1_body,
        grid=(_NT,),
        in_specs=[
            _tok_spec(_INP), _tok_spec(_INP),
            _tok_spec(2 * h), _tok_spec(2 * h),
            _rep_spec((_INP, 2 * h)), _rep_spec((1, 2 * h)),
            _rep_spec((_INP, 2 * h)), _rep_spec((1, 2 * h)),
            _rep_spec((2 * h, 2 * h)), _rep_spec((1, 2 * h)),
            _rep_spec((2 * h, h)), _rep_spec((1, h)),
            _rep_spec((2 * h, 2 * h)), _rep_spec((1, 2 * h)),
            _rep_spec((2 * h, h)), _rep_spec((1, h)),
            _rep_spec((h, h)), _rep_spec((1, h)),
            _rep_spec((h, 2 * h)), _rep_spec((1, 2 * h)),
            _rep_spec((h, h)), _rep_spec((1, h)),
            _rep_spec((h, 2 * h)), _rep_spec((1, 2 * h)),
            _rep_spec((2 * h, 2 * h)), _rep_spec((1, 2 * h)),
            _rep_spec((2 * h, 2 * h)), _rep_spec((1, 2 * h)),
        ],
        out_specs=[
            _tok_spec(2 * h), _tok_spec(h),
            pl.BlockSpec((1, 1, 2 * h), lambda i: (i, 0, 0)),
        ],
        out_shape=[
            jax.ShapeDtypeStruct((_T, 2 * h), f32),
            jax.ShapeDtypeStruct((_T, h), f32),
            jax.ShapeDtypeStruct((_NT, 1, 2 * h), f32),
        ],
        interpret=interpret,
    )(x1, x2, npack, dpack,
      wa, ba, wb, bb,
      wru1, bru1, p["g1_c_w"], p["g1_c_b"][None],
      wru2, bru2, p["g2_c_w"], p["g2_c_b"][None],
      p["proj1_w"], p["proj1_b"][None], wmv1, bmv1,
      p["proj2_w"], p["proj2_b"][None], wmv2, bmv2,
      wfv, bfv, sh, eh)

    out = pl.pallas_call(
        _pass2_body,
        grid=(_NT,),
        in_specs=[
            _tok_spec(2 * h), _tok_spec(h), _tok_spec(h),
            _rep_spec((_NT, 1, 2 * h)),
            _rep_spec((2 * h, 8)), _rep_spec((1, 8)),
            _rep_spec((h, _INP)), _rep_spec((1, _INP)),
        ],
        out_specs=[_tok_spec(_INP)],
        out_shape=[jax.ShapeDtypeStruct((_T, _INP), f32)],
        interpret=interpret,
    )(fmfv, var, eps, ps, qw, qb,
      p["proj_back_w"], p["proj_back_b"][None])[0]
    return out


def kernel(feature_1, feature_2, params):
    x1 = feature_1.reshape(_T, _INP)
    x2 = feature_2.reshape(_T, _INP)
    return _run(x1, x2, params).reshape(_B, _N, _INP)
